# baseline (speedup 1.0000x reference)
import dataclasses
import functools

import jax
import jax.numpy as jnp
from jax import lax
from jax.experimental import pallas as pl
from jax.experimental.pallas import tpu as pltpu
from jax.experimental.pallas import tpu_sc as plsc

N = 10000
E = 320000
IN_DIM = 128
HID = 32
OUT_DIM = 12
OUTP = 16

NC = 2
NS = 16
NW = NC * NS
K = 128
NBLK = 80
E_PAD = NW * NBLK * K
NP = 10240
RPS = NP // NS
TRS = N // NS

_sc_mesh = plsc.VectorSubcoreMesh(core_axis_name="c", subcore_axis_name="s")


def _rsqrt16(x):
    i = plsc.bitcast(x, jnp.int32)
    y = plsc.bitcast(jnp.int32(0x5F3759DF) - (i >> 1), jnp.float32)
    for _ in range(3):
        y = y * (1.5 - 0.5 * x * y * y)
    return y


def _make_edge_scatter(D, do_gather, scale_prologue=False):
    nbuf = 4 if do_gather else 2
    scratch = (
        [pltpu.VMEM((NBLK, K), jnp.int32)]
        + [pltpu.VMEM((K, D), jnp.float32)] * nbuf
        + [pltpu.VMEM_SHARED((NP, D), jnp.float32)]
        + [pltpu.SemaphoreType.DMA] * (2 * nbuf if do_gather else 2)
    )
    if do_gather:
        scratch.insert(0, pltpu.VMEM((NBLK, K), jnp.int32))
        scratch.append(pltpu.VMEM_SHARED((N, D), jnp.float32))
    if scale_prologue:
        scratch.append(pltpu.VMEM((TRS, D), jnp.float32))
        scratch.append(pltpu.VMEM((TRS, OUTP), jnp.float32))
        scratch.append(pltpu.VMEM((TRS, OUTP), jnp.float32))

    cp = pltpu.CompilerParams(use_tc_tiling_on_sc=False)
    if scale_prologue:
        cp = dataclasses.replace(cp, needs_layout_passes=False)

    @functools.partial(
        pl.kernel,
        out_type=jax.ShapeDtypeStruct((NC, NP, D), jnp.float32),
        mesh=_sc_mesh,
        scratch_types=scratch,
        compiler_params=cp,
    )
    def edge_scatter(*refs):
        if scale_prologue:
            (src_hbm, dst_hbm, table_hbm, degp_hbm, zeros_hbm, out_hbm,
             sidx, didx, *rest) = refs
            hbuf = rest[2 + 3 * nbuf]
            dbuf0 = rest[3 + 3 * nbuf]
            dbuf1 = rest[4 + 3 * nbuf]
        elif do_gather:
            (src_hbm, dst_hbm, table_hbm, zeros_hbm, out_hbm,
             sidx, didx, *rest) = refs
        else:
            (dst_hbm, ones_hbm, zeros_hbm, out_hbm,
             didx, rows0, rows1, acc, sem0, sem1) = refs
        if do_gather:
            rows = rest[:nbuf]
            acc = rest[nbuf]
            gsem = rest[nbuf + 1 : nbuf + 1 + nbuf]
            ssem = rest[nbuf + 1 + nbuf : 1 + 3 * nbuf]
            table_s = rest[1 + 3 * nbuf]

        c = lax.axis_index("c")
        s = lax.axis_index("s")
        wid = c * NS + s
        row0 = s * RPS
        pltpu.sync_copy(zeros_hbm.at[pl.ds(row0, RPS)], acc.at[pl.ds(row0, RPS)])
        pltpu.sync_copy(dst_hbm.at[wid], didx)
        if do_gather:
            pltpu.sync_copy(src_hbm.at[wid], sidx)
            trow0 = s * TRS
            if scale_prologue:
                pltpu.sync_copy(table_hbm.at[pl.ds(trow0, TRS)], hbuf)
                pltpu.sync_copy(degp_hbm.at[0, pl.ds(trow0, TRS)], dbuf0)
                pltpu.sync_copy(degp_hbm.at[1, pl.ds(trow0, TRS)], dbuf1)

                @pl.loop(0, TRS, step=25)
                def _(r):
                    ys = [_rsqrt16(dbuf0[r + u] + dbuf1[r + u] + 1.0)
                          for u in range(25)]
                    for u in range(25):
                        for h in range(D // 16):
                            sl = (r + u, pl.ds(16 * h, 16))
                            hbuf[sl] = ys[u] * hbuf[sl]

                pltpu.sync_copy(hbuf, table_s.at[pl.ds(trow0, TRS)])
            else:
                pltpu.sync_copy(table_hbm.at[pl.ds(trow0, TRS)],
                                table_s.at[pl.ds(trow0, TRS)])
        else:
            pltpu.sync_copy(ones_hbm, rows0)
        plsc.subcore_barrier()

        if do_gather:
            for t in range(nbuf):
                pltpu.async_copy(table_s.at[sidx.at[t]], rows[t], gsem[t])

            @pl.loop(0, NBLK - nbuf, step=nbuf)
            def _(j):
                for t in range(nbuf):
                    pltpu.make_async_copy(
                        table_s.at[sidx.at[j + t]], rows[t], gsem[t]).wait()
                    pltpu.async_copy(
                        rows[t], acc.at[didx.at[j + t]], ssem[t], add=True)
                for t in range(nbuf):
                    pltpu.make_async_copy(
                        rows[t], acc.at[didx.at[j + t]], ssem[t]).wait()
                    pltpu.async_copy(
                        table_s.at[sidx.at[j + nbuf + t]], rows[t], gsem[t])

            j0 = NBLK - nbuf
            for t in range(nbuf):
                pltpu.make_async_copy(
                    table_s.at[sidx.at[j0 + t]], rows[t], gsem[t]).wait()
                pltpu.async_copy(
                    rows[t], acc.at[didx.at[j0 + t]], ssem[t], add=True)
            for t in range(nbuf):
                pltpu.make_async_copy(
                    rows[t], acc.at[didx.at[j0 + t]], ssem[t]).wait()
        else:
            @pl.loop(0, NBLK, step=2)
            def _(j):
                pltpu.async_copy(rows0, acc.at[didx.at[j]], sem0, add=True)
                pltpu.async_copy(rows0, acc.at[didx.at[j + 1]], sem1, add=True)
                pltpu.make_async_copy(rows0, acc.at[didx.at[j]], sem0).wait()
                pltpu.make_async_copy(rows0, acc.at[didx.at[j + 1]], sem1).wait()

        plsc.subcore_barrier()
        pltpu.sync_copy(acc.at[pl.ds(row0, RPS)],
                        out_hbm.at[c, pl.ds(row0, RPS)])

    return edge_scatter


_deg_kernel = _make_edge_scatter(OUTP, do_gather=False)
_agg32_kernel = _make_edge_scatter(HID, do_gather=True, scale_prologue=True)
_agg16_kernel = _make_edge_scatter(OUTP, do_gather=True)


def _tc1_body(x_ref, w1_ref, out_ref):
    out_ref[...] = jnp.dot(x_ref[...], w1_ref[...],
                           preferred_element_type=jnp.float32)


def _tcdis_body(degp_ref, out_ref):
    degp = degp_ref[...]
    out_ref[...] = lax.rsqrt(degp[0, :N, 0:1] + degp[1, :N, 0:1] + 1.0)


def _tc2_body(dis_ref, aggp_ref, h1_ref, b1_ref, w2_ref, out_ref):
    dis = dis_ref[...]
    aggp = aggp_ref[...]
    out1 = (dis * (aggp[0, :N] + aggp[1, :N]) + (dis * dis) * h1_ref[...]
            + b1_ref[...])
    h = jnp.maximum(out1, 0.0)
    h2 = jnp.dot(h, w2_ref[...], preferred_element_type=jnp.float32)
    out_ref[...] = dis * h2


def _tc3_body(dis_ref, aggp_ref, scaled2_ref, b2_ref, out_ref):
    dis = dis_ref[...]
    aggp = aggp_ref[...]
    full = dis * (aggp[0, :N] + aggp[1, :N] + scaled2_ref[...])
    out_ref[...] = full[:, :OUT_DIM] + b2_ref[...]


def kernel(x, adj, W1, b1, W2, b2):
    src = adj[0]
    dst = adj[1]

    pad = E_PAD - E
    src3d = jnp.concatenate(
        [src, jnp.zeros((pad,), jnp.int32)]).reshape(NW, NBLK, K)
    dst3d = jnp.concatenate(
        [dst, N + (jnp.arange(pad, dtype=jnp.int32) % (NP - N))]
    ).reshape(NW, NBLK, K)

    ones_blk = jnp.ones((K, OUTP), dtype=jnp.float32)
    zeros16 = jnp.zeros((NP, OUTP), dtype=jnp.float32)
    zeros32 = jnp.zeros((NP, HID), dtype=jnp.float32)
    w2p = jnp.zeros((HID, OUTP), dtype=jnp.float32).at[:, : W2.shape[1]].set(W2)
    b1r = b1.reshape(1, HID)
    b2r = b2.reshape(1, OUT_DIM)

    degp = _deg_kernel(dst3d, ones_blk, zeros16)
    h1 = pl.pallas_call(
        _tc1_body,
        out_shape=jax.ShapeDtypeStruct((N, HID), jnp.float32),
    )(x, W1)

    aggp1 = _agg32_kernel(src3d, dst3d, h1, degp, zeros32)

    dis = pl.pallas_call(
        _tcdis_body,
        out_shape=jax.ShapeDtypeStruct((N, 1), jnp.float32),
    )(degp)

    scaled2 = pl.pallas_call(
        _tc2_body,
        out_shape=jax.ShapeDtypeStruct((N, OUTP), jnp.float32),
    )(dis, aggp1, h1, b1r, w2p)

    aggp2 = _agg16_kernel(src3d, dst3d, scaled2, zeros16)

    out = pl.pallas_call(
        _tc3_body,
        out_shape=jax.ShapeDtypeStruct((N, OUT_DIM), jnp.float32),
    )(dis, aggp2, scaled2, b2r)

    return out


# speedup vs baseline: 1.0098x; 1.0098x over previous
import dataclasses
import functools

import jax
import jax.numpy as jnp
from jax import lax
from jax.experimental import pallas as pl
from jax.experimental.pallas import tpu as pltpu
from jax.experimental.pallas import tpu_sc as plsc

N = 10000
E = 320000
IN_DIM = 128
HID = 32
OUT_DIM = 12
OUTP = 16

NC = 2
NS = 16
NW = NC * NS
K = 128
NBLK = 80
E_PAD = NW * NBLK * K
NP = 10240
RPS = NP // NS
TRS = N // NS

_sc_mesh = plsc.VectorSubcoreMesh(core_axis_name="c", subcore_axis_name="s")


def _rsqrt16(x):
    i = plsc.bitcast(x, jnp.int32)
    y = plsc.bitcast(jnp.int32(0x5F3759DF) - (i >> 1), jnp.float32)
    for _ in range(3):
        y = y * (1.5 - 0.5 * x * y * y)
    return y


def _make_edge_scatter(D, do_gather, scale_prologue=False):
    nbuf = 4 if do_gather else 2
    scratch = (
        [pltpu.VMEM((NBLK, K), jnp.int32)]
        + [pltpu.VMEM((K, D), jnp.float32)] * nbuf
        + [pltpu.VMEM_SHARED((NP, D), jnp.float32)]
        + [pltpu.SemaphoreType.DMA] * (2 * nbuf if do_gather else 2)
    )
    if do_gather:
        scratch.insert(0, pltpu.VMEM((NBLK, K), jnp.int32))
        scratch.append(pltpu.VMEM_SHARED((N, D), jnp.float32))
    if scale_prologue:
        scratch.append(pltpu.VMEM((TRS, D), jnp.float32))
        scratch.append(pltpu.VMEM((TRS, OUTP), jnp.float32))
        scratch.append(pltpu.VMEM((TRS, OUTP), jnp.float32))

    cp = pltpu.CompilerParams(use_tc_tiling_on_sc=False)
    if scale_prologue:
        cp = dataclasses.replace(cp, needs_layout_passes=False)

    @functools.partial(
        pl.kernel,
        out_type=jax.ShapeDtypeStruct((NC, NP, D), jnp.float32),
        mesh=_sc_mesh,
        scratch_types=scratch,
        compiler_params=cp,
    )
    def edge_scatter(*refs):
        if scale_prologue:
            (src_hbm, dst_hbm, table_hbm, degp_hbm, zeros_hbm, out_hbm,
             sidx, didx, *rest) = refs
            hbuf = rest[2 + 3 * nbuf]
            dbuf0 = rest[3 + 3 * nbuf]
            dbuf1 = rest[4 + 3 * nbuf]
        elif do_gather:
            (src_hbm, dst_hbm, table_hbm, zeros_hbm, out_hbm,
             sidx, didx, *rest) = refs
        else:
            (dst_hbm, ones_hbm, zeros_hbm, out_hbm,
             didx, rows0, rows1, acc, sem0, sem1) = refs
        if do_gather:
            rows = rest[:nbuf]
            acc = rest[nbuf]
            gsem = rest[nbuf + 1 : nbuf + 1 + nbuf]
            ssem = rest[nbuf + 1 + nbuf : 1 + 3 * nbuf]
            table_s = rest[1 + 3 * nbuf]

        c = lax.axis_index("c")
        s = lax.axis_index("s")
        wid = c * NS + s
        row0 = s * RPS
        pltpu.sync_copy(zeros_hbm.at[pl.ds(row0, RPS)], acc.at[pl.ds(row0, RPS)])
        pltpu.sync_copy(dst_hbm.at[wid], didx)
        if do_gather:
            pltpu.sync_copy(src_hbm.at[wid], sidx)
            trow0 = s * TRS
            if scale_prologue:
                pltpu.sync_copy(table_hbm.at[pl.ds(trow0, TRS)], hbuf)
                pltpu.sync_copy(degp_hbm.at[0, pl.ds(trow0, TRS)], dbuf0)
                pltpu.sync_copy(degp_hbm.at[1, pl.ds(trow0, TRS)], dbuf1)

                @pl.loop(0, TRS, step=5)
                def _(r):
                    ys = [_rsqrt16(dbuf0[r + u] + dbuf1[r + u] + 1.0)
                          for u in range(5)]
                    for u in range(5):
                        for h in range(D // 16):
                            sl = (r + u, pl.ds(16 * h, 16))
                            hbuf[sl] = ys[u] * hbuf[sl]

                pltpu.sync_copy(hbuf, table_s.at[pl.ds(trow0, TRS)])
            else:
                pltpu.sync_copy(table_hbm.at[pl.ds(trow0, TRS)],
                                table_s.at[pl.ds(trow0, TRS)])
        else:
            pltpu.sync_copy(ones_hbm, rows0)
        plsc.subcore_barrier()

        if do_gather:
            for t in range(nbuf):
                pltpu.async_copy(table_s.at[sidx.at[t]], rows[t], gsem[t])

            @pl.loop(0, NBLK - nbuf, step=nbuf)
            def _(j):
                for t in range(nbuf):
                    pltpu.make_async_copy(
                        table_s.at[sidx.at[j + t]], rows[t], gsem[t]).wait()
                    pltpu.async_copy(
                        rows[t], acc.at[didx.at[j + t]], ssem[t], add=True)
                for t in range(nbuf):
                    pltpu.make_async_copy(
                        rows[t], acc.at[didx.at[j + t]], ssem[t]).wait()
                    pltpu.async_copy(
                        table_s.at[sidx.at[j + nbuf + t]], rows[t], gsem[t])

            j0 = NBLK - nbuf
            for t in range(nbuf):
                pltpu.make_async_copy(
                    table_s.at[sidx.at[j0 + t]], rows[t], gsem[t]).wait()
                pltpu.async_copy(
                    rows[t], acc.at[didx.at[j0 + t]], ssem[t], add=True)
            for t in range(nbuf):
                pltpu.make_async_copy(
                    rows[t], acc.at[didx.at[j0 + t]], ssem[t]).wait()
        else:
            @pl.loop(0, NBLK, step=2)
            def _(j):
                pltpu.async_copy(rows0, acc.at[didx.at[j]], sem0, add=True)
                pltpu.async_copy(rows0, acc.at[didx.at[j + 1]], sem1, add=True)
                pltpu.make_async_copy(rows0, acc.at[didx.at[j]], sem0).wait()
                pltpu.make_async_copy(rows0, acc.at[didx.at[j + 1]], sem1).wait()

        plsc.subcore_barrier()
        pltpu.sync_copy(acc.at[pl.ds(row0, RPS)],
                        out_hbm.at[c, pl.ds(row0, RPS)])

    return edge_scatter


_deg_kernel = _make_edge_scatter(OUTP, do_gather=False)
_agg32_kernel = _make_edge_scatter(HID, do_gather=True, scale_prologue=True)
_agg16_kernel = _make_edge_scatter(OUTP, do_gather=True)


def _tc1_body(x_ref, w1_ref, out_ref):
    out_ref[...] = jnp.dot(x_ref[...], w1_ref[...],
                           preferred_element_type=jnp.float32)


def _tcdis_body(degp_ref, out_ref):
    degp = degp_ref[...]
    out_ref[...] = lax.rsqrt(degp[0, :N, 0:1] + degp[1, :N, 0:1] + 1.0)


def _tc2_body(dis_ref, aggp_ref, h1_ref, b1_ref, w2_ref, out_ref):
    dis = dis_ref[...]
    aggp = aggp_ref[...]
    out1 = (dis * (aggp[0, :N] + aggp[1, :N]) + (dis * dis) * h1_ref[...]
            + b1_ref[...])
    h = jnp.maximum(out1, 0.0)
    h2 = jnp.dot(h, w2_ref[...], preferred_element_type=jnp.float32)
    out_ref[...] = dis * h2


def _tc3_body(dis_ref, aggp_ref, scaled2_ref, b2_ref, out_ref):
    dis = dis_ref[...]
    aggp = aggp_ref[...]
    full = dis * (aggp[0, :N] + aggp[1, :N] + scaled2_ref[...])
    out_ref[...] = full[:, :OUT_DIM] + b2_ref[...]


def kernel(x, adj, W1, b1, W2, b2):
    src = adj[0]
    dst = adj[1]

    pad = E_PAD - E
    src3d = jnp.concatenate(
        [src, jnp.zeros((pad,), jnp.int32)]).reshape(NW, NBLK, K)
    dst3d = jnp.concatenate(
        [dst, N + (jnp.arange(pad, dtype=jnp.int32) % (NP - N))]
    ).reshape(NW, NBLK, K)

    ones_blk = jnp.ones((K, OUTP), dtype=jnp.float32)
    zeros16 = jnp.zeros((NP, OUTP), dtype=jnp.float32)
    zeros32 = jnp.zeros((NP, HID), dtype=jnp.float32)
    w2p = jnp.zeros((HID, OUTP), dtype=jnp.float32).at[:, : W2.shape[1]].set(W2)
    b1r = b1.reshape(1, HID)
    b2r = b2.reshape(1, OUT_DIM)

    degp = _deg_kernel(dst3d, ones_blk, zeros16)
    h1 = pl.pallas_call(
        _tc1_body,
        out_shape=jax.ShapeDtypeStruct((N, HID), jnp.float32),
    )(x, W1)

    aggp1 = _agg32_kernel(src3d, dst3d, h1, degp, zeros32)

    dis = pl.pallas_call(
        _tcdis_body,
        out_shape=jax.ShapeDtypeStruct((N, 1), jnp.float32),
    )(degp)

    scaled2 = pl.pallas_call(
        _tc2_body,
        out_shape=jax.ShapeDtypeStruct((N, OUTP), jnp.float32),
    )(dis, aggp1, h1, b1r, w2p)

    aggp2 = _agg16_kernel(src3d, dst3d, scaled2, zeros16)

    out = pl.pallas_call(
        _tc3_body,
        out_shape=jax.ShapeDtypeStruct((N, OUT_DIM), jnp.float32),
    )(dis, aggp2, scaled2, b2r)

    return out
